# Initial kernel scaffold; baseline (speedup 1.0000x reference)
import functools

import jax
import jax.numpy as jnp
from jax import lax
from jax.experimental import pallas as pl
from jax.experimental.pallas import tpu as pltpu
from jax.experimental.pallas import tpu_sc as plsc

N = 10000
E = 320000
F = 128
K = 64

NUM_CORES = 2
NUM_SUBCORES = 16
NUM_TILES = NUM_CORES * NUM_SUBCORES
EDGES_PER_TILE = E // NUM_TILES
B = 80
BATCHES = EDGES_PER_TILE // B
ROWS_PER_TILE = N // NUM_SUBCORES
RCHUNK = 125
LANES = 16
VPR = F // LANES


_BE = 4000


def _tc_a_body(rbf_ref, x_ref, wk2f_ref, wi_ref, bi_ref, wj_ref, bj_ref,
               g_ref, xi_ref, dj_ref):
    g_ref[...] = jnp.dot(rbf_ref[...], wk2f_ref[...],
                         preferred_element_type=jnp.float32)

    @pl.when(pl.program_id(0) == 0)
    def _():
        xv = x_ref[...]
        xi_ref[...] = jnp.dot(xv, wi_ref[...],
                              preferred_element_type=jnp.float32) + bi_ref[...]
        dj_ref[...] = jnp.dot(xv, wj_ref[...],
                              preferred_element_type=jnp.float32) + bj_ref[...]


def _tc_a(rbf, x, wk2f, wi, bi, wj, bj):
    grid = (E // _BE,)
    full = lambda shape: pl.BlockSpec(shape, lambda i: (0, 0))
    return pl.pallas_call(
        _tc_a_body,
        grid=grid,
        in_specs=[
            pl.BlockSpec((_BE, K), lambda i: (i, 0)),
            full((N, F)),
            full((K, F)),
            full((F, F)),
            full((1, F)),
            full((F, F)),
            full((1, F)),
        ],
        out_specs=[
            pl.BlockSpec((_BE, F), lambda i: (i, 0)),
            full((N, F)),
            full((N, F)),
        ],
        out_shape=[
            jax.ShapeDtypeStruct((E, F), jnp.float32),
            jax.ShapeDtypeStruct((N, F), jnp.float32),
            jax.ShapeDtypeStruct((N, F), jnp.float32),
        ],
    )(rbf, x, wk2f, wi, bi, wj, bj)


def _sc_body(g_hbm, idx_i_hbm, idx_j_hbm, dj_hbm, out_hbm,
             idxi_v, idxj_v, g_v, rows_v, zbuf, xj_sh, sem):
    c = lax.axis_index("c")
    s = lax.axis_index("s")
    wid = c * NUM_SUBCORES + s
    edge_base = wid * EDGES_PER_TILE

    zero = jnp.zeros((LANES,), jnp.float32)

    def zero_body(e, _):
        for cc in range(VPR):
            zbuf[e, pl.ds(cc * LANES, LANES)] = zero
        return 0

    lax.fori_loop(0, RCHUNK, zero_body, 0)

    row0 = s * ROWS_PER_TILE
    for j in range(ROWS_PER_TILE // RCHUNK):
        pltpu.sync_copy(zbuf, xj_sh.at[pl.ds(row0 + j * RCHUNK, RCHUNK)])

    plsc.subcore_barrier()

    def batch_body(it, _):
        base = edge_base + it * B
        pltpu.sync_copy(idx_i_hbm.at[pl.ds(base, B)], idxi_v)
        pltpu.sync_copy(idx_j_hbm.at[pl.ds(base, B)], idxj_v)
        pltpu.sync_copy(g_hbm.at[pl.ds(base, B)], g_v)
        pltpu.async_copy(dj_hbm.at[idxj_v], rows_v, sem).wait()

        def mul_body(e, _):
            for cc in range(VPR):
                sl = pl.ds(cc * LANES, LANES)
                rows_v[e, sl] = rows_v[e, sl] * g_v[e, sl]
            return 0

        lax.fori_loop(0, B, mul_body, 0)

        pltpu.sync_copy(rows_v, xj_sh.at[idxi_v], add=True)
        return 0

    lax.fori_loop(0, BATCHES, batch_body, 0)

    plsc.subcore_barrier()

    out_base = c * N + row0
    for j in range(ROWS_PER_TILE // RCHUNK):
        pltpu.sync_copy(xj_sh.at[pl.ds(row0 + j * RCHUNK, RCHUNK)], zbuf)
        pltpu.sync_copy(zbuf, out_hbm.at[pl.ds(out_base + j * RCHUNK, RCHUNK)])


def _sc_segment_sum(g, idx_i, idx_j, dj):
    mesh = plsc.VectorSubcoreMesh(core_axis_name="c", subcore_axis_name="s")
    f = pl.kernel(
        _sc_body,
        out_type=jax.ShapeDtypeStruct((NUM_CORES * N, F), jnp.float32),
        mesh=mesh,
        scratch_types=[
            pltpu.VMEM((B,), jnp.int32),
            pltpu.VMEM((B,), jnp.int32),
            pltpu.VMEM((B, F), jnp.float32),
            pltpu.VMEM((B, F), jnp.float32),
            pltpu.VMEM((RCHUNK, F), jnp.float32),
            pltpu.VMEM_SHARED((N, F), jnp.float32),
            pltpu.SemaphoreType.DMA,
        ],
    )
    return f(g, idx_i, idx_j, dj)


_BN = 1000


def _tc_b_body(x_ref, xi_ref, p0_ref, p1_ref, wr1_ref, br1_ref, wr2_ref,
               br2_ref, wd_ref, bd_ref, u_ref, out_ref):
    m = xi_ref[...] + p0_ref[...] + p1_ref[...]
    h = jnp.dot(m, wr1_ref[...], preferred_element_type=jnp.float32) + br1_ref[...]
    m = m + jnp.dot(h, wr2_ref[...], preferred_element_type=jnp.float32) + br2_ref[...]
    out_ref[...] = (u_ref[...] * x_ref[...]
                    + jnp.dot(m, wd_ref[...], preferred_element_type=jnp.float32)
                    + bd_ref[...])


def _tc_b(x, xi, p0, p1, wr1, br1, wr2, br2, wd, bd, u):
    grid = (N // _BN,)
    rows = pl.BlockSpec((_BN, F), lambda i: (i, 0))
    full = lambda shape: pl.BlockSpec(shape, lambda i: (0, 0))
    return pl.pallas_call(
        _tc_b_body,
        grid=grid,
        in_specs=[rows, rows, rows, rows,
                  full((F, F)), full((1, F)), full((F, F)), full((1, F)),
                  full((F, F)), full((1, F)), full((1, F))],
        out_specs=rows,
        out_shape=jax.ShapeDtypeStruct((N, F), jnp.float32),
    )(x, xi, p0, p1, wr1, br1, wr2, br2, wd, bd, u)


def kernel(x, rbf, idx_i, idx_j, Wk2f, Wi, bi, Wj, bj, Wr1, br1, Wr2, br2,
           Wd, bd, u):
    bi2 = bi.reshape(1, F)
    bj2 = bj.reshape(1, F)
    br12 = br1.reshape(1, F)
    br22 = br2.reshape(1, F)
    bd2 = bd.reshape(1, F)
    u2 = u.reshape(1, F)

    g, xi, dj = _tc_a(rbf, x, Wk2f, Wi, bi2, Wj, bj2)
    partials = _sc_segment_sum(g, idx_i.astype(jnp.int32),
                               idx_j.astype(jnp.int32), dj)
    p0 = partials[:N]
    p1 = partials[N:]
    return _tc_b(x, xi, p0, p1, Wr1, br12, Wr2, br22, Wd, bd2, u2)


# speedup vs baseline: 2.4342x; 2.4342x over previous
import functools

import jax
import jax.numpy as jnp
from jax import lax
from jax.experimental import pallas as pl
from jax.experimental.pallas import tpu as pltpu
from jax.experimental.pallas import tpu_sc as plsc

N = 10000
E = 320000
F = 128
K = 64

NUM_CORES = 2
NUM_SUBCORES = 16
NUM_TILES = NUM_CORES * NUM_SUBCORES
EDGES_PER_TILE = E // NUM_TILES
B = 80
BATCHES = EDGES_PER_TILE // B
NPAD = 10240
ROWS_PER_TILE = NPAD // NUM_SUBCORES
RCHUNK = 128
LANES = 16
VPR = F // LANES


_BE = 4000


def _tc_a_body(rbf_ref, x_ref, wk2f_ref, wi_ref, bi_ref, wj_ref, bj_ref,
               g_ref, xi_ref, dj_ref):
    g_ref[...] = jnp.dot(rbf_ref[...], wk2f_ref[...],
                         preferred_element_type=jnp.float32)

    @pl.when(pl.program_id(0) == 0)
    def _():
        xv = x_ref[...]
        xi_ref[...] = jnp.dot(xv, wi_ref[...],
                              preferred_element_type=jnp.float32) + bi_ref[...]
        dj_ref[...] = jnp.dot(xv, wj_ref[...],
                              preferred_element_type=jnp.float32) + bj_ref[...]


def _tc_a(rbf, x, wk2f, wi, bi, wj, bj):
    grid = (E // _BE,)
    full = lambda shape: pl.BlockSpec(shape, lambda i: (0, 0))
    return pl.pallas_call(
        _tc_a_body,
        grid=grid,
        in_specs=[
            pl.BlockSpec((_BE, K), lambda i: (i, 0)),
            full((N, F)),
            full((K, F)),
            full((F, F)),
            full((1, F)),
            full((F, F)),
            full((1, F)),
        ],
        out_specs=[
            pl.BlockSpec((_BE, F), lambda i: (i, 0)),
            full((N, F)),
            full((N, F)),
        ],
        out_shape=[
            jax.ShapeDtypeStruct((E, F), jnp.float32),
            jax.ShapeDtypeStruct((N, F), jnp.float32),
            jax.ShapeDtypeStruct((N, F), jnp.float32),
        ],
    )(rbf, x, wk2f, wi, bi, wj, bj)


def _sc_body(g_hbm, idx_i_hbm, idx_j_hbm, dj_hbm, out_hbm,
             idxi_v, idxj_v, g_v, rows_v, zbuf, xj_sh, sem):
    c = lax.axis_index("c")
    s = lax.axis_index("s")
    wid = c * NUM_SUBCORES + s
    edge_base = wid * EDGES_PER_TILE

    zero = jnp.zeros((LANES,), jnp.float32)

    def zero_body(e, _):
        for cc in range(VPR):
            zbuf[e, pl.ds(cc * LANES, LANES)] = zero
        return 0

    lax.fori_loop(0, RCHUNK, zero_body, 0)

    row0 = s * ROWS_PER_TILE
    for j in range(ROWS_PER_TILE // RCHUNK):
        pltpu.sync_copy(zbuf, xj_sh.at[pl.ds(row0 + j * RCHUNK, RCHUNK)])

    plsc.subcore_barrier()

    def batch_body(it, _):
        base = edge_base + it * B
        pltpu.sync_copy(idx_i_hbm.at[pl.ds(base, B)], idxi_v)
        pltpu.sync_copy(idx_j_hbm.at[pl.ds(base, B)], idxj_v)
        pltpu.sync_copy(g_hbm.at[pl.ds(base, B)], g_v)
        pltpu.async_copy(dj_hbm.at[idxj_v], rows_v, sem).wait()

        def mul_body(e, _):
            for cc in range(VPR):
                sl = pl.ds(cc * LANES, LANES)
                rows_v[e, sl] = rows_v[e, sl] * g_v[e, sl]
            return 0

        lax.fori_loop(0, B, mul_body, 0)

        pltpu.sync_copy(rows_v, xj_sh.at[idxi_v], add=True)
        return 0

    lax.fori_loop(0, BATCHES, batch_body, 0)

    plsc.subcore_barrier()

    out_base = c * NPAD + row0
    for j in range(ROWS_PER_TILE // RCHUNK):
        pltpu.sync_copy(xj_sh.at[pl.ds(row0 + j * RCHUNK, RCHUNK)], zbuf)
        pltpu.sync_copy(zbuf, out_hbm.at[pl.ds(out_base + j * RCHUNK, RCHUNK)])


def _sc_segment_sum(g, idx_i, idx_j, dj):
    mesh = plsc.VectorSubcoreMesh(core_axis_name="c", subcore_axis_name="s")
    f = pl.kernel(
        _sc_body,
        out_type=jax.ShapeDtypeStruct((NUM_CORES * NPAD, F), jnp.float32),
        mesh=mesh,
        scratch_types=[
            pltpu.VMEM((B,), jnp.int32),
            pltpu.VMEM((B,), jnp.int32),
            pltpu.VMEM((B, F), jnp.float32),
            pltpu.VMEM((B, F), jnp.float32),
            pltpu.VMEM((RCHUNK, F), jnp.float32),
            pltpu.VMEM_SHARED((NPAD, F), jnp.float32),
            pltpu.SemaphoreType.DMA,
        ],
    )
    return f(g, idx_i, idx_j, dj)


_BN = 1000


def _tc_b_body(x_ref, xi_ref, p0_ref, p1_ref, wr1_ref, br1_ref, wr2_ref,
               br2_ref, wd_ref, bd_ref, u_ref, out_ref):
    m = xi_ref[...] + p0_ref[...] + p1_ref[...]
    h = jnp.dot(m, wr1_ref[...], preferred_element_type=jnp.float32) + br1_ref[...]
    m = m + jnp.dot(h, wr2_ref[...], preferred_element_type=jnp.float32) + br2_ref[...]
    out_ref[...] = (u_ref[...] * x_ref[...]
                    + jnp.dot(m, wd_ref[...], preferred_element_type=jnp.float32)
                    + bd_ref[...])


def _tc_b(x, xi, p0, p1, wr1, br1, wr2, br2, wd, bd, u):
    grid = (N // _BN,)
    rows = pl.BlockSpec((_BN, F), lambda i: (i, 0))
    full = lambda shape: pl.BlockSpec(shape, lambda i: (0, 0))
    return pl.pallas_call(
        _tc_b_body,
        grid=grid,
        in_specs=[rows, rows, rows, rows,
                  full((F, F)), full((1, F)), full((F, F)), full((1, F)),
                  full((F, F)), full((1, F)), full((1, F))],
        out_specs=rows,
        out_shape=jax.ShapeDtypeStruct((N, F), jnp.float32),
    )(x, xi, p0, p1, wr1, br1, wr2, br2, wd, bd, u)


def kernel(x, rbf, idx_i, idx_j, Wk2f, Wi, bi, Wj, bj, Wr1, br1, Wr2, br2,
           Wd, bd, u):
    bi2 = bi.reshape(1, F)
    bj2 = bj.reshape(1, F)
    br12 = br1.reshape(1, F)
    br22 = br2.reshape(1, F)
    bd2 = bd.reshape(1, F)
    u2 = u.reshape(1, F)

    g, xi, dj = _tc_a(rbf, x, Wk2f, Wi, bi2, Wj, bj2)
    partials = _sc_segment_sum(g, idx_i.astype(jnp.int32),
                               idx_j.astype(jnp.int32), dj)
    p0 = partials[:N]
    p1 = partials[NPAD:NPAD + N]
    return _tc_b(x, xi, p0, p1, Wr1, br12, Wr2, br22, Wd, bd2, u2)


# speedup vs baseline: 3.9421x; 1.6194x over previous
import functools

import jax
import jax.numpy as jnp
from jax import lax
from jax.experimental import pallas as pl
from jax.experimental.pallas import tpu as pltpu
from jax.experimental.pallas import tpu_sc as plsc

N = 10000
E = 320000
F = 128
K = 64

NUM_CORES = 2
NUM_SUBCORES = 16
NUM_TILES = NUM_CORES * NUM_SUBCORES
EDGES_PER_TILE = E // NUM_TILES
B = 80
BATCHES = EDGES_PER_TILE // B
NPAD = 10240
ROWS_PER_TILE = NPAD // NUM_SUBCORES
RCHUNK = 128
LANES = 16
VPR = F // LANES


_BE = 4000


def _tc_a_body(rbf_ref, x_ref, wk2f_ref, wi_ref, bi_ref, wj_ref, bj_ref,
               g_ref, xi_ref, dj_ref):
    g_ref[...] = jnp.dot(rbf_ref[...], wk2f_ref[...],
                         preferred_element_type=jnp.float32)

    @pl.when(pl.program_id(0) == 0)
    def _():
        xv = x_ref[...]
        xi_ref[...] = jnp.dot(xv, wi_ref[...],
                              preferred_element_type=jnp.float32) + bi_ref[...]
        dj_ref[...] = jnp.dot(xv, wj_ref[...],
                              preferred_element_type=jnp.float32) + bj_ref[...]


def _tc_a(rbf, x, wk2f, wi, bi, wj, bj):
    grid = (E // _BE,)
    full = lambda shape: pl.BlockSpec(shape, lambda i: (0, 0))
    return pl.pallas_call(
        _tc_a_body,
        grid=grid,
        in_specs=[
            pl.BlockSpec((_BE, K), lambda i: (i, 0)),
            full((N, F)),
            full((K, F)),
            full((F, F)),
            full((1, F)),
            full((F, F)),
            full((1, F)),
        ],
        out_specs=[
            pl.BlockSpec((_BE, F), lambda i: (i, 0)),
            full((N, F)),
            full((N, F)),
        ],
        out_shape=[
            jax.ShapeDtypeStruct((E, F), jnp.float32),
            jax.ShapeDtypeStruct((N, F), jnp.float32),
            jax.ShapeDtypeStruct((N, F), jnp.float32),
        ],
    )(rbf, x, wk2f, wi, bi, wj, bj)


def _sc_body(g_hbm, idx_i_hbm, idx_j_hbm, dj_hbm, out_hbm,
             idxi_v0, idxi_v1, idxj_v0, idxj_v1, g_v0, g_v1,
             rows_v0, rows_v1, xj_sh,
             sem_i0, sem_i1, sem_j0, sem_j1, sem_g0, sem_g1, sem_r0, sem_r1):
    c = lax.axis_index("c")
    s = lax.axis_index("s")
    wid = c * NUM_SUBCORES + s
    edge_base = wid * EDGES_PER_TILE

    idxi_v = (idxi_v0, idxi_v1)
    idxj_v = (idxj_v0, idxj_v1)
    g_v = (g_v0, g_v1)
    rows_v = (rows_v0, rows_v1)
    sem_i = (sem_i0, sem_i1)
    sem_j = (sem_j0, sem_j1)
    sem_g = (sem_g0, sem_g1)
    sem_r = (sem_r0, sem_r1)

    zero = jnp.zeros((LANES,), jnp.float32)

    def zero_body(e, _):
        for cc in range(VPR):
            g_v0[e, pl.ds(cc * LANES, LANES)] = zero
        return 0

    lax.fori_loop(0, B, zero_body, 0)

    row0 = s * ROWS_PER_TILE
    for j in range(ROWS_PER_TILE // B):
        off = row0 + j * B

        @pl.when(off + B <= N)
        def _():
            pltpu.sync_copy(g_v0, xj_sh.at[pl.ds(off, B)])

    plsc.subcore_barrier()

    def issue(it, sl):
        base = edge_base + it * B
        pltpu.async_copy(idx_i_hbm.at[pl.ds(base, B)], idxi_v[sl], sem_i[sl])
        pltpu.async_copy(idx_j_hbm.at[pl.ds(base, B)], idxj_v[sl], sem_j[sl])
        pltpu.async_copy(g_hbm.at[pl.ds(base, B)], g_v[sl], sem_g[sl])
        pltpu.make_async_copy(idx_j_hbm.at[pl.ds(base, B)],
                              idxj_v[sl], sem_j[sl]).wait()
        pltpu.async_copy(dj_hbm.at[idxj_v[sl]], rows_v[sl], sem_r[sl])

    def proc(it, sl):
        @pl.when(it + 1 < BATCHES)
        def _():
            issue(it + 1, 1 - sl)

        base = edge_base + it * B
        pltpu.make_async_copy(g_hbm.at[pl.ds(base, B)],
                              g_v[sl], sem_g[sl]).wait()
        pltpu.make_async_copy(dj_hbm.at[idxj_v[sl]],
                              rows_v[sl], sem_r[sl]).wait()

        def mul_body(e, _):
            for cc in range(VPR):
                ds = pl.ds(cc * LANES, LANES)
                rows_v[sl][e, ds] = rows_v[sl][e, ds] * g_v[sl][e, ds]
            return 0

        lax.fori_loop(0, B, mul_body, 0)

        pltpu.make_async_copy(idx_i_hbm.at[pl.ds(base, B)],
                              idxi_v[sl], sem_i[sl]).wait()
        pltpu.sync_copy(rows_v[sl], xj_sh.at[idxi_v[sl]], add=True)

    issue(0, 0)

    def pair(i2, _):
        it0 = i2 * 2
        proc(it0, 0)
        proc(it0 + 1, 1)
        return 0

    lax.fori_loop(0, BATCHES // 2, pair, 0)
    proc(BATCHES - 1, 0)

    plsc.subcore_barrier()

    for j in range(ROWS_PER_TILE // B):
        off = row0 + j * B

        @pl.when(off + B <= N)
        def _():
            pltpu.sync_copy(xj_sh.at[pl.ds(off, B)], rows_v0)
            pltpu.sync_copy(rows_v0, out_hbm.at[pl.ds(c * N + off, B)])


def _sc_segment_sum(g, idx_i, idx_j, dj):
    mesh = plsc.VectorSubcoreMesh(core_axis_name="c", subcore_axis_name="s")
    f = pl.kernel(
        _sc_body,
        out_type=jax.ShapeDtypeStruct((NUM_CORES * N, F), jnp.float32),
        mesh=mesh,
        scratch_types=[
            pltpu.VMEM((B,), jnp.int32),
            pltpu.VMEM((B,), jnp.int32),
            pltpu.VMEM((B,), jnp.int32),
            pltpu.VMEM((B,), jnp.int32),
            pltpu.VMEM((B, F), jnp.float32),
            pltpu.VMEM((B, F), jnp.float32),
            pltpu.VMEM((B, F), jnp.float32),
            pltpu.VMEM((B, F), jnp.float32),
            pltpu.VMEM_SHARED((N, F), jnp.float32),
            pltpu.SemaphoreType.DMA,
            pltpu.SemaphoreType.DMA,
            pltpu.SemaphoreType.DMA,
            pltpu.SemaphoreType.DMA,
            pltpu.SemaphoreType.DMA,
            pltpu.SemaphoreType.DMA,
            pltpu.SemaphoreType.DMA,
            pltpu.SemaphoreType.DMA,
        ],
    )
    return f(g, idx_i, idx_j, dj)


_BN = 1000


def _tc_b_body(x_ref, xi_ref, p0_ref, p1_ref, wr1_ref, br1_ref, wr2_ref,
               br2_ref, wd_ref, bd_ref, u_ref, out_ref):
    m = xi_ref[...] + p0_ref[...] + p1_ref[...]
    h = jnp.dot(m, wr1_ref[...], preferred_element_type=jnp.float32) + br1_ref[...]
    m = m + jnp.dot(h, wr2_ref[...], preferred_element_type=jnp.float32) + br2_ref[...]
    out_ref[...] = (u_ref[...] * x_ref[...]
                    + jnp.dot(m, wd_ref[...], preferred_element_type=jnp.float32)
                    + bd_ref[...])


def _tc_b(x, xi, p0, p1, wr1, br1, wr2, br2, wd, bd, u):
    grid = (N // _BN,)
    rows = pl.BlockSpec((_BN, F), lambda i: (i, 0))
    full = lambda shape: pl.BlockSpec(shape, lambda i: (0, 0))
    return pl.pallas_call(
        _tc_b_body,
        grid=grid,
        in_specs=[rows, rows, rows, rows,
                  full((F, F)), full((1, F)), full((F, F)), full((1, F)),
                  full((F, F)), full((1, F)), full((1, F))],
        out_specs=rows,
        out_shape=jax.ShapeDtypeStruct((N, F), jnp.float32),
    )(x, xi, p0, p1, wr1, br1, wr2, br2, wd, bd, u)


def kernel(x, rbf, idx_i, idx_j, Wk2f, Wi, bi, Wj, bj, Wr1, br1, Wr2, br2,
           Wd, bd, u):
    bi2 = bi.reshape(1, F)
    bj2 = bj.reshape(1, F)
    br12 = br1.reshape(1, F)
    br22 = br2.reshape(1, F)
    bd2 = bd.reshape(1, F)
    u2 = u.reshape(1, F)

    g, xi, dj = _tc_a(rbf, x, Wk2f, Wi, bi2, Wj, bj2)
    partials = _sc_segment_sum(g, idx_i.astype(jnp.int32),
                               idx_j.astype(jnp.int32), dj)
    p0 = partials[:N]
    p1 = partials[N:]
    return _tc_b(x, xi, p0, p1, Wr1, br12, Wr2, br22, Wd, bd2, u2)


# speedup vs baseline: 4.5093x; 1.1439x over previous
import jax
import jax.numpy as jnp
from jax import lax
from jax.experimental import pallas as pl
from jax.experimental.pallas import tpu as pltpu
from jax.experimental.pallas import tpu_sc as plsc

N = 10000
E = 320000
F = 128
K = 64

NUM_CORES = 2
NUM_SUBCORES = 16
NUM_TILES = NUM_CORES * NUM_SUBCORES
HALF_E = E // 2
PAIRS_PER_TILE = HALF_E // NUM_TILES
B2 = 40
B = 2 * B2
BATCHES = PAIRS_PER_TILE // B2
ROWS_PER_TILE = 640
LANES = 16
VPR = F // LANES
HMASK = -65536


_BE = 4000


def _tc_a_body(rbf_lo_ref, rbf_hi_ref, x_ref, wk2f_ref, wi_ref, bi_ref,
               wj_ref, bj_ref, g_ref, xi_ref, dj_ref):
    wk = wk2f_ref[...]
    gl = jnp.dot(rbf_lo_ref[...], wk, preferred_element_type=jnp.float32)
    gh = jnp.dot(rbf_hi_ref[...], wk, preferred_element_type=jnp.float32)
    pe = jax.lax.bitcast_convert_type(gl.astype(jnp.bfloat16),
                                      jnp.uint16).astype(jnp.uint32)
    po = jax.lax.bitcast_convert_type(gh.astype(jnp.bfloat16),
                                      jnp.uint16).astype(jnp.uint32)
    g_ref[...] = jax.lax.bitcast_convert_type(pe | (po << 16), jnp.int32)

    @pl.when(pl.program_id(0) == 0)
    def _():
        xv = x_ref[...]
        xi_ref[...] = jnp.dot(xv, wi_ref[...],
                              preferred_element_type=jnp.float32) + bi_ref[...]
        dj_ref[...] = jnp.dot(xv, wj_ref[...],
                              preferred_element_type=jnp.float32) + bj_ref[...]


def _tc_a(rbf, x, wk2f, wi, bi, wj, bj):
    grid = (HALF_E // _BE,)
    full = lambda shape: pl.BlockSpec(shape, lambda i: (0, 0))
    nhi = HALF_E // _BE
    return pl.pallas_call(
        _tc_a_body,
        grid=grid,
        in_specs=[
            pl.BlockSpec((_BE, K), lambda i: (i, 0)),
            pl.BlockSpec((_BE, K), lambda i: (i + nhi, 0)),
            full((N, F)),
            full((K, F)),
            full((F, F)),
            full((1, F)),
            full((F, F)),
            full((1, F)),
        ],
        out_specs=[
            pl.BlockSpec((_BE, F), lambda i: (i, 0)),
            full((N, F)),
            full((N, F)),
        ],
        out_shape=[
            jax.ShapeDtypeStruct((HALF_E, F), jnp.int32),
            jax.ShapeDtypeStruct((N, F), jnp.float32),
            jax.ShapeDtypeStruct((N, F), jnp.float32),
        ],
    )(rbf, rbf, x, wk2f, wi, bi, wj, bj)


def _sc_body(g_hbm, idx_i_hbm, idx_j_hbm, dj_hbm, out_hbm,
             idxi_v0, idxi_v1, idxjl_v0, idxjl_v1, idxjh_v0, idxjh_v1,
             g_v0, g_v1, rows_v0, rows_v1, msg_v, xj_sh,
             sem_i0, sem_i1, sem_jl0, sem_jl1, sem_jh0, sem_jh1,
             sem_g0, sem_g1, sem_r0, sem_r1):
    c = lax.axis_index("c")
    s = lax.axis_index("s")
    wid = c * NUM_SUBCORES + s
    pair_base = wid * PAIRS_PER_TILE

    idxi_v = (idxi_v0, idxi_v1)
    idxjl_v = (idxjl_v0, idxjl_v1)
    idxjh_v = (idxjh_v0, idxjh_v1)
    g_v = (g_v0, g_v1)
    rows_v = (rows_v0, rows_v1)
    sem_i = (sem_i0, sem_i1)
    sem_jl = (sem_jl0, sem_jl1)
    sem_jh = (sem_jh0, sem_jh1)
    sem_g = (sem_g0, sem_g1)
    sem_r = (sem_r0, sem_r1)

    zero = jnp.zeros((LANES,), jnp.float32)

    def zero_body(e, _):
        for cc in range(VPR):
            msg_v[e, pl.ds(cc * LANES, LANES)] = zero
        return 0

    lax.fori_loop(0, B, zero_body, 0)

    row0 = s * ROWS_PER_TILE
    for j in range(ROWS_PER_TILE // B):
        off = row0 + j * B

        @pl.when(off + B <= N)
        def _():
            pltpu.sync_copy(msg_v, xj_sh.at[pl.ds(off, B)])

    plsc.subcore_barrier()

    def issue(it, sl):
        blo = pair_base + it * B2
        bhi = blo + HALF_E
        pltpu.async_copy(idx_i_hbm.at[pl.ds(blo, B2)],
                         idxi_v[sl].at[pl.ds(0, B2)], sem_i[sl])
        pltpu.async_copy(idx_i_hbm.at[pl.ds(bhi, B2)],
                         idxi_v[sl].at[pl.ds(B2, B2)], sem_i[sl])
        pltpu.async_copy(idx_j_hbm.at[pl.ds(blo, B2)], idxjl_v[sl], sem_jl[sl])
        pltpu.async_copy(idx_j_hbm.at[pl.ds(bhi, B2)], idxjh_v[sl], sem_jh[sl])
        pltpu.async_copy(g_hbm.at[pl.ds(blo, B2)], g_v[sl], sem_g[sl])
        pltpu.make_async_copy(idx_j_hbm.at[pl.ds(blo, B2)],
                              idxjl_v[sl], sem_jl[sl]).wait()
        pltpu.async_copy(dj_hbm.at[idxjl_v[sl]],
                         rows_v[sl].at[pl.ds(0, B2)], sem_r[sl])
        pltpu.make_async_copy(idx_j_hbm.at[pl.ds(bhi, B2)],
                              idxjh_v[sl], sem_jh[sl]).wait()
        pltpu.async_copy(dj_hbm.at[idxjh_v[sl]],
                         rows_v[sl].at[pl.ds(B2, B2)], sem_r[sl])

    def proc(it, sl):
        @pl.when(it + 1 < BATCHES)
        def _():
            issue(it + 1, 1 - sl)

        blo = pair_base + it * B2
        bhi = blo + HALF_E
        pltpu.make_async_copy(g_hbm.at[pl.ds(blo, B2)],
                              g_v[sl], sem_g[sl]).wait()
        pltpu.make_async_copy(dj_hbm.at[idxjl_v[sl]],
                              rows_v[sl].at[pl.ds(0, B2)], sem_r[sl]).wait()
        pltpu.make_async_copy(dj_hbm.at[idxjh_v[sl]],
                              rows_v[sl].at[pl.ds(B2, B2)], sem_r[sl]).wait()

        def mul_body(e2, _):
            for q in range(VPR):
                ds = pl.ds(q * LANES, LANES)
                wg = g_v[sl][e2, ds]
                g0 = jax.lax.bitcast_convert_type(wg << 16, jnp.float32)
                g1 = jax.lax.bitcast_convert_type(wg & HMASK, jnp.float32)
                msg_v[e2, ds] = g0 * rows_v[sl][e2, ds]
                msg_v[e2 + B2, ds] = g1 * rows_v[sl][e2 + B2, ds]
            return 0

        lax.fori_loop(0, B2, mul_body, 0)

        pltpu.make_async_copy(idx_i_hbm.at[pl.ds(blo, B2)],
                              idxi_v[sl].at[pl.ds(0, B2)], sem_i[sl]).wait()
        pltpu.make_async_copy(idx_i_hbm.at[pl.ds(bhi, B2)],
                              idxi_v[sl].at[pl.ds(B2, B2)], sem_i[sl]).wait()
        pltpu.sync_copy(msg_v, xj_sh.at[idxi_v[sl]], add=True)

    issue(0, 0)

    def pair(i2, _):
        it0 = i2 * 2
        proc(it0, 0)
        proc(it0 + 1, 1)
        return 0

    lax.fori_loop(0, BATCHES // 2, pair, 0)
    proc(BATCHES - 1, 0)

    plsc.subcore_barrier()

    for j in range(ROWS_PER_TILE // B):
        off = row0 + j * B

        @pl.when(off + B <= N)
        def _():
            pltpu.sync_copy(xj_sh.at[pl.ds(off, B)], msg_v)
            pltpu.sync_copy(msg_v, out_hbm.at[pl.ds(c * N + off, B)])


def _sc_segment_sum(g, idx_i, idx_j, dj):
    mesh = plsc.VectorSubcoreMesh(core_axis_name="c", subcore_axis_name="s")
    f = pl.kernel(
        _sc_body,
        out_type=jax.ShapeDtypeStruct((NUM_CORES * N, F), jnp.float32),
        mesh=mesh,
        scratch_types=[
            pltpu.VMEM((B,), jnp.int32),
            pltpu.VMEM((B,), jnp.int32),
            pltpu.VMEM((B2,), jnp.int32),
            pltpu.VMEM((B2,), jnp.int32),
            pltpu.VMEM((B2,), jnp.int32),
            pltpu.VMEM((B2,), jnp.int32),
            pltpu.VMEM((B2, F), jnp.int32),
            pltpu.VMEM((B2, F), jnp.int32),
            pltpu.VMEM((B, F), jnp.float32),
            pltpu.VMEM((B, F), jnp.float32),
            pltpu.VMEM((B, F), jnp.float32),
            pltpu.VMEM_SHARED((N, F), jnp.float32),
            pltpu.SemaphoreType.DMA,
            pltpu.SemaphoreType.DMA,
            pltpu.SemaphoreType.DMA,
            pltpu.SemaphoreType.DMA,
            pltpu.SemaphoreType.DMA,
            pltpu.SemaphoreType.DMA,
            pltpu.SemaphoreType.DMA,
            pltpu.SemaphoreType.DMA,
            pltpu.SemaphoreType.DMA,
            pltpu.SemaphoreType.DMA,
        ],
    )
    return f(g, idx_i, idx_j, dj)


_BN = 1000


def _tc_b_body(x_ref, xi_ref, p0_ref, p1_ref, wr1_ref, br1_ref, wr2_ref,
               br2_ref, wd_ref, bd_ref, u_ref, out_ref):
    m = xi_ref[...] + p0_ref[...] + p1_ref[...]
    h = jnp.dot(m, wr1_ref[...], preferred_element_type=jnp.float32) + br1_ref[...]
    m = m + jnp.dot(h, wr2_ref[...], preferred_element_type=jnp.float32) + br2_ref[...]
    out_ref[...] = (u_ref[...] * x_ref[...]
                    + jnp.dot(m, wd_ref[...], preferred_element_type=jnp.float32)
                    + bd_ref[...])


def _tc_b(x, xi, p0, p1, wr1, br1, wr2, br2, wd, bd, u):
    grid = (N // _BN,)
    rows = pl.BlockSpec((_BN, F), lambda i: (i, 0))
    full = lambda shape: pl.BlockSpec(shape, lambda i: (0, 0))
    return pl.pallas_call(
        _tc_b_body,
        grid=grid,
        in_specs=[rows, rows, rows, rows,
                  full((F, F)), full((1, F)), full((F, F)), full((1, F)),
                  full((F, F)), full((1, F)), full((1, F))],
        out_specs=rows,
        out_shape=jax.ShapeDtypeStruct((N, F), jnp.float32),
    )(x, xi, p0, p1, wr1, br1, wr2, br2, wd, bd, u)


def kernel(x, rbf, idx_i, idx_j, Wk2f, Wi, bi, Wj, bj, Wr1, br1, Wr2, br2,
           Wd, bd, u):
    bi2 = bi.reshape(1, F)
    bj2 = bj.reshape(1, F)
    br12 = br1.reshape(1, F)
    br22 = br2.reshape(1, F)
    bd2 = bd.reshape(1, F)
    u2 = u.reshape(1, F)

    g, xi, dj = _tc_a(rbf, x, Wk2f, Wi, bi2, Wj, bj2)
    partials = _sc_segment_sum(g, idx_i.astype(jnp.int32),
                               idx_j.astype(jnp.int32), dj)
    p0 = partials[:N]
    p1 = partials[N:]
    return _tc_b(x, xi, p0, p1, Wr1, br12, Wr2, br22, Wd, bd2, u2)


# speedup vs baseline: 6.3888x; 1.4168x over previous
import jax
import jax.numpy as jnp
from jax import lax
from jax.experimental import pallas as pl
from jax.experimental.pallas import tpu as pltpu
from jax.experimental.pallas import tpu_sc as plsc

N = 10000
E = 320000
F = 128
K = 64

NUM_CORES = 2
NUM_SUBCORES = 16
NUM_TILES = NUM_CORES * NUM_SUBCORES
HALF_E = E // 2
PAIRS_PER_TILE = HALF_E // NUM_TILES
B2 = 40
B = 2 * B2
BATCHES = PAIRS_PER_TILE // B2
ROWS_PER_TILE = 640
LANES = 16
VPR = F // LANES
HMASK = -65536


_BE = 6400


_DN_T = (((0,), (0,)), ((), ()))


def _tc_a_body(rbf_lo_ref, rbf_hi_ref, x_ref, wk2f_ref,
               wj_ref, bj_ref, g_ref, dj_ref):
    wk = wk2f_ref[...]
    gl = jax.lax.dot_general(rbf_lo_ref[...], wk, _DN_T,
                             preferred_element_type=jnp.float32)
    gh = jax.lax.dot_general(rbf_hi_ref[...], wk, _DN_T,
                             preferred_element_type=jnp.float32)
    pe = jax.lax.bitcast_convert_type(gl.astype(jnp.bfloat16),
                                      jnp.uint16).astype(jnp.uint32)
    po = jax.lax.bitcast_convert_type(gh.astype(jnp.bfloat16),
                                      jnp.uint16).astype(jnp.uint32)
    g_ref[...] = jax.lax.bitcast_convert_type(pe | (po << 16), jnp.int32)

    @pl.when(pl.program_id(0) == 0)
    def _():
        dj_ref[...] = jnp.dot(x_ref[...], wj_ref[...],
                              preferred_element_type=jnp.float32) + bj_ref[...]


def _tc_a(rbf_t, x, wk2f, wj, bj):
    grid = (HALF_E // _BE,)
    full = lambda shape: pl.BlockSpec(shape, lambda i: (0, 0))
    nhi = HALF_E // _BE
    return pl.pallas_call(
        _tc_a_body,
        grid=grid,
        in_specs=[
            pl.BlockSpec((K, _BE), lambda i: (0, i)),
            pl.BlockSpec((K, _BE), lambda i: (0, i + nhi)),
            full((N, F)),
            full((K, F)),
            full((F, F)),
            full((1, F)),
        ],
        out_specs=[
            pl.BlockSpec((_BE, F), lambda i: (i, 0)),
            full((N, F)),
        ],
        out_shape=[
            jax.ShapeDtypeStruct((HALF_E, F), jnp.int32),
            jax.ShapeDtypeStruct((N, F), jnp.float32),
        ],
    )(rbf_t, rbf_t, x, wk2f, wj, bj)


def _sc_body(g_hbm, idx_i_hbm, idx_j_hbm, dj_hbm, out_hbm,
             idxi_v0, idxi_v1, idxjl_v0, idxjl_v1, idxjh_v0, idxjh_v1,
             g_v0, g_v1, rows_v0, rows_v1, xj_sh,
             sem_i0, sem_i1, sem_jl0, sem_jl1, sem_jh0, sem_jh1,
             sem_g0, sem_g1, sem_r0, sem_r1, sem_s0, sem_s1):
    c = lax.axis_index("c")
    s = lax.axis_index("s")
    wid = c * NUM_SUBCORES + s
    pair_base = wid * PAIRS_PER_TILE

    idxi_v = (idxi_v0, idxi_v1)
    idxjl_v = (idxjl_v0, idxjl_v1)
    idxjh_v = (idxjh_v0, idxjh_v1)
    g_v = (g_v0, g_v1)
    rows_v = (rows_v0, rows_v1)
    sem_i = (sem_i0, sem_i1)
    sem_jl = (sem_jl0, sem_jl1)
    sem_jh = (sem_jh0, sem_jh1)
    sem_g = (sem_g0, sem_g1)
    sem_r = (sem_r0, sem_r1)
    sem_s = (sem_s0, sem_s1)

    zero = jnp.zeros((LANES,), jnp.float32)

    def zero_body(e, _):
        for cc in range(VPR):
            rows_v0[e, pl.ds(cc * LANES, LANES)] = zero
        return 0

    lax.fori_loop(0, B, zero_body, 0)

    row0 = s * ROWS_PER_TILE
    for j in range(ROWS_PER_TILE // B):
        off = row0 + j * B

        @pl.when(off + B <= N)
        def _():
            pltpu.sync_copy(rows_v0, xj_sh.at[pl.ds(off, B)])

    plsc.subcore_barrier()

    def issue(it, sl):
        blo = pair_base + it * B2
        bhi = blo + HALF_E

        @pl.when(it >= 2)
        def _():
            pltpu.make_async_copy(rows_v[sl], xj_sh.at[idxi_v[sl]],
                                  sem_s[sl]).wait()

        pltpu.async_copy(idx_i_hbm.at[pl.ds(blo, B2)],
                         idxi_v[sl].at[pl.ds(0, B2)], sem_i[sl])
        pltpu.async_copy(idx_i_hbm.at[pl.ds(bhi, B2)],
                         idxi_v[sl].at[pl.ds(B2, B2)], sem_i[sl])
        pltpu.async_copy(idx_j_hbm.at[pl.ds(blo, B2)], idxjl_v[sl], sem_jl[sl])
        pltpu.async_copy(idx_j_hbm.at[pl.ds(bhi, B2)], idxjh_v[sl], sem_jh[sl])
        pltpu.async_copy(g_hbm.at[pl.ds(blo, B2)], g_v[sl], sem_g[sl])
        pltpu.make_async_copy(idx_j_hbm.at[pl.ds(blo, B2)],
                              idxjl_v[sl], sem_jl[sl]).wait()
        pltpu.async_copy(dj_hbm.at[idxjl_v[sl]],
                         rows_v[sl].at[pl.ds(0, B2)], sem_r[sl])
        pltpu.make_async_copy(idx_j_hbm.at[pl.ds(bhi, B2)],
                              idxjh_v[sl], sem_jh[sl]).wait()
        pltpu.async_copy(dj_hbm.at[idxjh_v[sl]],
                         rows_v[sl].at[pl.ds(B2, B2)], sem_r[sl])

    def proc(it, sl):
        @pl.when(it + 1 < BATCHES)
        def _():
            issue(it + 1, 1 - sl)

        blo = pair_base + it * B2
        bhi = blo + HALF_E
        pltpu.make_async_copy(g_hbm.at[pl.ds(blo, B2)],
                              g_v[sl], sem_g[sl]).wait()
        pltpu.make_async_copy(dj_hbm.at[idxjl_v[sl]],
                              rows_v[sl].at[pl.ds(0, B2)], sem_r[sl]).wait()
        pltpu.make_async_copy(dj_hbm.at[idxjh_v[sl]],
                              rows_v[sl].at[pl.ds(B2, B2)], sem_r[sl]).wait()

        def mul_body(e2, _):
            for q in range(VPR):
                ds = pl.ds(q * LANES, LANES)
                wg = g_v[sl][e2, ds]
                g0 = jax.lax.bitcast_convert_type(wg << 16, jnp.float32)
                g1 = jax.lax.bitcast_convert_type(wg & HMASK, jnp.float32)
                rows_v[sl][e2, ds] = g0 * rows_v[sl][e2, ds]
                rows_v[sl][e2 + B2, ds] = g1 * rows_v[sl][e2 + B2, ds]
            return 0

        lax.fori_loop(0, B2, mul_body, 0)

        pltpu.make_async_copy(idx_i_hbm.at[pl.ds(blo, B2)],
                              idxi_v[sl].at[pl.ds(0, B2)], sem_i[sl]).wait()
        pltpu.make_async_copy(idx_i_hbm.at[pl.ds(bhi, B2)],
                              idxi_v[sl].at[pl.ds(B2, B2)], sem_i[sl]).wait()
        pltpu.async_copy(rows_v[sl], xj_sh.at[idxi_v[sl]], sem_s[sl],
                         add=True)

    issue(0, 0)

    def pair(i2, _):
        it0 = i2 * 2
        proc(it0, 0)
        proc(it0 + 1, 1)
        return 0

    lax.fori_loop(0, BATCHES // 2, pair, 0)
    proc(BATCHES - 1, 0)

    pltpu.make_async_copy(rows_v[1], xj_sh.at[idxi_v[1]], sem_s[1]).wait()
    pltpu.make_async_copy(rows_v[0], xj_sh.at[idxi_v[0]], sem_s[0]).wait()

    plsc.subcore_barrier()

    for j in range(ROWS_PER_TILE // B):
        off = row0 + j * B

        @pl.when(off + B <= N)
        def _():
            pltpu.sync_copy(xj_sh.at[pl.ds(off, B)], rows_v0)
            pltpu.sync_copy(rows_v0, out_hbm.at[pl.ds(c * N + off, B)])


def _sc_segment_sum(g, idx_i, idx_j, dj):
    mesh = plsc.VectorSubcoreMesh(core_axis_name="c", subcore_axis_name="s")
    f = pl.kernel(
        _sc_body,
        out_type=jax.ShapeDtypeStruct((NUM_CORES * N, F), jnp.float32),
        mesh=mesh,
        scratch_types=[
            pltpu.VMEM((B,), jnp.int32),
            pltpu.VMEM((B,), jnp.int32),
            pltpu.VMEM((B2,), jnp.int32),
            pltpu.VMEM((B2,), jnp.int32),
            pltpu.VMEM((B2,), jnp.int32),
            pltpu.VMEM((B2,), jnp.int32),
            pltpu.VMEM((B2, F), jnp.int32),
            pltpu.VMEM((B2, F), jnp.int32),
            pltpu.VMEM((B, F), jnp.float32),
            pltpu.VMEM((B, F), jnp.float32),
            pltpu.VMEM_SHARED((N, F), jnp.float32),
            pltpu.SemaphoreType.DMA,
            pltpu.SemaphoreType.DMA,
            pltpu.SemaphoreType.DMA,
            pltpu.SemaphoreType.DMA,
            pltpu.SemaphoreType.DMA,
            pltpu.SemaphoreType.DMA,
            pltpu.SemaphoreType.DMA,
            pltpu.SemaphoreType.DMA,
            pltpu.SemaphoreType.DMA,
            pltpu.SemaphoreType.DMA,
            pltpu.SemaphoreType.DMA,
            pltpu.SemaphoreType.DMA,
        ],
    )
    return f(g, idx_i, idx_j, dj)


_BN = 1000


def _tc_b_body(x_ref, p0_ref, p1_ref, wi_ref, bi_ref, wr1_ref, br1_ref,
               wr2_ref, br2_ref, wd_ref, bd_ref, u_ref, out_ref):
    xv = x_ref[...]
    m = (jnp.dot(xv, wi_ref[...], preferred_element_type=jnp.float32)
         + bi_ref[...] + p0_ref[...] + p1_ref[...])
    h = jnp.dot(m, wr1_ref[...], preferred_element_type=jnp.float32) + br1_ref[...]
    m = m + jnp.dot(h, wr2_ref[...], preferred_element_type=jnp.float32) + br2_ref[...]
    out_ref[...] = (u_ref[...] * xv
                    + jnp.dot(m, wd_ref[...], preferred_element_type=jnp.float32)
                    + bd_ref[...])


def _tc_b(x, p0, p1, wi, bi, wr1, br1, wr2, br2, wd, bd, u):
    grid = (N // _BN,)
    rows = pl.BlockSpec((_BN, F), lambda i: (i, 0))
    full = lambda shape: pl.BlockSpec(shape, lambda i: (0, 0))
    return pl.pallas_call(
        _tc_b_body,
        grid=grid,
        in_specs=[rows, rows, rows,
                  full((F, F)), full((1, F)), full((F, F)), full((1, F)),
                  full((F, F)), full((1, F)), full((F, F)), full((1, F)),
                  full((1, F))],
        out_specs=rows,
        out_shape=jax.ShapeDtypeStruct((N, F), jnp.float32),
    )(x, p0, p1, wi, bi, wr1, br1, wr2, br2, wd, bd, u)


def kernel(x, rbf, idx_i, idx_j, Wk2f, Wi, bi, Wj, bj, Wr1, br1, Wr2, br2,
           Wd, bd, u):
    bi2 = bi.reshape(1, F)
    bj2 = bj.reshape(1, F)
    br12 = br1.reshape(1, F)
    br22 = br2.reshape(1, F)
    bd2 = bd.reshape(1, F)
    u2 = u.reshape(1, F)

    g, dj = _tc_a(rbf.T, x, Wk2f, Wj, bj2)
    partials = _sc_segment_sum(g, idx_i.astype(jnp.int32),
                               idx_j.astype(jnp.int32), dj)
    p0 = partials[:N]
    p1 = partials[N:]
    return _tc_b(x, p0, p1, Wi, bi2, Wr1, br12, Wr2, br22, Wd, bd2, u2)


# speedup vs baseline: 7.0310x; 1.1005x over previous
import jax
import jax.numpy as jnp
from jax import lax
from jax.experimental import pallas as pl
from jax.experimental.pallas import tpu as pltpu
from jax.experimental.pallas import tpu_sc as plsc

N = 10000
E = 320000
F = 128
K = 64

NUM_CORES = 2
NUM_SUBCORES = 16
NUM_TILES = NUM_CORES * NUM_SUBCORES
HALF_E = E // 2
PAIRS_PER_TILE = HALF_E // NUM_TILES
B2 = 40
B = 2 * B2
BATCHES = PAIRS_PER_TILE // B2
ROWS_PER_TILE = 640
LANES = 16
VPR = F // LANES
HMASK = -65536


_BE = 6400


_DN_T = (((0,), (0,)), ((), ()))


def _tc_a_body(rbf_lo_ref, rbf_hi_ref, x_ref, wk2f_ref,
               wj_ref, bj_ref, g_ref, dj_ref):
    wk = wk2f_ref[...]
    gl = jax.lax.dot_general(rbf_lo_ref[...], wk, _DN_T,
                             preferred_element_type=jnp.float32)
    gh = jax.lax.dot_general(rbf_hi_ref[...], wk, _DN_T,
                             preferred_element_type=jnp.float32)
    pe = jax.lax.bitcast_convert_type(gl.astype(jnp.bfloat16),
                                      jnp.uint16).astype(jnp.uint32)
    po = jax.lax.bitcast_convert_type(gh.astype(jnp.bfloat16),
                                      jnp.uint16).astype(jnp.uint32)
    g_ref[...] = jax.lax.bitcast_convert_type(pe | (po << 16), jnp.int32)

    @pl.when(pl.program_id(0) == 0)
    def _():
        dj_ref[...] = jnp.dot(x_ref[...], wj_ref[...],
                              preferred_element_type=jnp.float32) + bj_ref[...]


def _tc_a(rbf_t, x, wk2f, wj, bj):
    grid = (HALF_E // _BE,)
    full = lambda shape: pl.BlockSpec(shape, lambda i: (0, 0))
    nhi = HALF_E // _BE
    return pl.pallas_call(
        _tc_a_body,
        grid=grid,
        in_specs=[
            pl.BlockSpec((K, _BE), lambda i: (0, i)),
            pl.BlockSpec((K, _BE), lambda i: (0, i + nhi)),
            full((N, F)),
            full((K, F)),
            full((F, F)),
            full((1, F)),
        ],
        out_specs=[
            pl.BlockSpec((_BE, F), lambda i: (i, 0)),
            full((N, F)),
        ],
        out_shape=[
            jax.ShapeDtypeStruct((HALF_E, F), jnp.int32),
            jax.ShapeDtypeStruct((N, F), jnp.float32),
        ],
    )(rbf_t, rbf_t, x, wk2f, wj, bj)


def _sc_body(g_hbm, idx_i_hbm, idx_j_hbm, dj_hbm, out_hbm,
             idxi_v0, idxi_v1, idxi_v2, idxjl_v0, idxjl_v1,
             idxjh_v0, idxjh_v1, g_v0, g_v1, rows_v0, rows_v1, rows_v2,
             xj_sh,
             sem_i0, sem_i1, sem_i2, sem_jl0, sem_jl1, sem_jh0, sem_jh1,
             sem_g0, sem_g1, sem_r0, sem_r1, sem_r2, sem_s0, sem_s1, sem_s2):
    c = lax.axis_index("c")
    s = lax.axis_index("s")
    wid = c * NUM_SUBCORES + s
    pair_base = wid * PAIRS_PER_TILE

    idxi_v = (idxi_v0, idxi_v1, idxi_v2)
    idxjl_v = (idxjl_v0, idxjl_v1)
    idxjh_v = (idxjh_v0, idxjh_v1)
    g_v = (g_v0, g_v1)
    rows_v = (rows_v0, rows_v1, rows_v2)
    sem_i = (sem_i0, sem_i1, sem_i2)
    sem_jl = (sem_jl0, sem_jl1)
    sem_jh = (sem_jh0, sem_jh1)
    sem_g = (sem_g0, sem_g1)
    sem_r = (sem_r0, sem_r1, sem_r2)
    sem_s = (sem_s0, sem_s1, sem_s2)

    zero = jnp.zeros((LANES,), jnp.float32)

    def zero_body(e, _):
        for cc in range(VPR):
            rows_v0[e, pl.ds(cc * LANES, LANES)] = zero
        return 0

    lax.fori_loop(0, B, zero_body, 0)

    row0 = s * ROWS_PER_TILE
    for j in range(ROWS_PER_TILE // B):
        off = row0 + j * B

        @pl.when(off + B <= N)
        def _():
            pltpu.sync_copy(rows_v0, xj_sh.at[pl.ds(off, B)])

    plsc.subcore_barrier()


    def issue_idxj(it, js):
        blo = pair_base + it * B2
        bhi = blo + HALF_E
        pltpu.async_copy(idx_j_hbm.at[pl.ds(blo, B2)], idxjl_v[js],
                         sem_jl[js])
        pltpu.async_copy(idx_j_hbm.at[pl.ds(bhi, B2)], idxjh_v[js],
                         sem_jh[js])

    def issue_main(it, rs, gs):
        blo = pair_base + it * B2
        bhi = blo + HALF_E
        pltpu.async_copy(idx_i_hbm.at[pl.ds(blo, B2)],
                         idxi_v[rs].at[pl.ds(0, B2)], sem_i[rs])
        pltpu.async_copy(idx_i_hbm.at[pl.ds(bhi, B2)],
                         idxi_v[rs].at[pl.ds(B2, B2)], sem_i[rs])
        pltpu.async_copy(g_hbm.at[pl.ds(blo, B2)], g_v[gs], sem_g[gs])
        pltpu.make_async_copy(idx_j_hbm.at[pl.ds(blo, B2)],
                              idxjl_v[gs], sem_jl[gs]).wait()
        pltpu.async_copy(dj_hbm.at[idxjl_v[gs]],
                         rows_v[rs].at[pl.ds(0, B2)], sem_r[rs])
        pltpu.make_async_copy(idx_j_hbm.at[pl.ds(bhi, B2)],
                              idxjh_v[gs], sem_jh[gs]).wait()
        pltpu.async_copy(dj_hbm.at[idxjh_v[gs]],
                         rows_v[rs].at[pl.ds(B2, B2)], sem_r[rs])

    def wait_scatter(sl):
        pltpu.make_async_copy(rows_v[sl], xj_sh.at[idxi_v[sl]],
                              sem_s[sl]).wait()

    def body_of(it, rs, gs, blo, bhi):
        pltpu.make_async_copy(g_hbm.at[pl.ds(blo, B2)],
                              g_v[gs], sem_g[gs]).wait()
        pltpu.make_async_copy(dj_hbm.at[idxjl_v[gs]],
                              rows_v[rs].at[pl.ds(0, B2)], sem_r[rs]).wait()
        pltpu.make_async_copy(dj_hbm.at[idxjh_v[gs]],
                              rows_v[rs].at[pl.ds(B2, B2)], sem_r[rs]).wait()

        @plsc.parallel_loop(0, B2, 1, unroll=2)
        def mul_body(e2):
            for q in range(VPR):
                ds = pl.ds(q * LANES, LANES)
                wg = g_v[gs][e2, ds]
                g0 = jax.lax.bitcast_convert_type(wg << 16, jnp.float32)
                g1 = jax.lax.bitcast_convert_type(wg & HMASK, jnp.float32)
                rows_v[rs][e2, ds] = g0 * rows_v[rs][e2, ds]
                rows_v[rs][e2 + B2, ds] = g1 * rows_v[rs][e2 + B2, ds]

        pltpu.make_async_copy(idx_i_hbm.at[pl.ds(blo, B2)],
                              idxi_v[rs].at[pl.ds(0, B2)], sem_i[rs]).wait()
        pltpu.make_async_copy(idx_i_hbm.at[pl.ds(bhi, B2)],
                              idxi_v[rs].at[pl.ds(B2, B2)], sem_i[rs]).wait()
        pltpu.async_copy(rows_v[rs], xj_sh.at[idxi_v[rs]], sem_s[rs],
                         add=True)

    issue_idxj(0, 0)
    issue_idxj(1, 1)
    issue_main(0, 0, 0)

    def six(i6, _):
        it0 = i6 * 6
        for kk in range(6):
            it = it0 + kk
            rs, gs = kk % 3, kk % 2
            rs_n, gs_n = (rs + 1) % 3, (gs + 1) % 2
            blo = pair_base + it * B2
            bhi = blo + HALF_E

            issue_idxj(it + 2, gs)

            @pl.when(it >= 2)
            def _(rs_n=rs_n):
                wait_scatter(rs_n)

            issue_main(it + 1, rs_n, gs_n)
            body_of(it, rs, gs, blo, bhi)
        return 0

    lax.fori_loop(0, (BATCHES - 5) // 6, six, 0)

    for it in range(BATCHES - 5, BATCHES):
        rs, gs = it % 3, it % 2
        rs_n, gs_n = (rs + 1) % 3, (gs + 1) % 2
        blo = pair_base + it * B2
        bhi = blo + HALF_E

        if it + 2 < BATCHES:
            issue_idxj(it + 2, gs)
        if it + 1 < BATCHES:
            wait_scatter(rs_n)
            issue_main(it + 1, rs_n, gs_n)
        body_of(it, rs, gs, blo, bhi)

    wait_scatter((BATCHES - 3) % 3)
    wait_scatter((BATCHES - 2) % 3)
    wait_scatter((BATCHES - 1) % 3)

    plsc.subcore_barrier()

    for j in range(ROWS_PER_TILE // B):
        off = row0 + j * B

        @pl.when(off + B <= N)
        def _():
            pltpu.sync_copy(xj_sh.at[pl.ds(off, B)], rows_v0)
            pltpu.sync_copy(rows_v0, out_hbm.at[pl.ds(c * N + off, B)])


def _sc_segment_sum(g, idx_i, idx_j, dj):
    mesh = plsc.VectorSubcoreMesh(core_axis_name="c", subcore_axis_name="s")
    f = pl.kernel(
        _sc_body,
        out_type=jax.ShapeDtypeStruct((NUM_CORES * N, F), jnp.float32),
        mesh=mesh,
        scratch_types=[
            pltpu.VMEM((B,), jnp.int32),
            pltpu.VMEM((B,), jnp.int32),
            pltpu.VMEM((B,), jnp.int32),
            pltpu.VMEM((B2,), jnp.int32),
            pltpu.VMEM((B2,), jnp.int32),
            pltpu.VMEM((B2,), jnp.int32),
            pltpu.VMEM((B2,), jnp.int32),
            pltpu.VMEM((B2, F), jnp.int32),
            pltpu.VMEM((B2, F), jnp.int32),
            pltpu.VMEM((B, F), jnp.float32),
            pltpu.VMEM((B, F), jnp.float32),
            pltpu.VMEM((B, F), jnp.float32),
            pltpu.VMEM_SHARED((N, F), jnp.float32),
        ] + [pltpu.SemaphoreType.DMA] * 15,
    )
    return f(g, idx_i, idx_j, dj)


_BN = 1000


def _tc_b_body(x_ref, p0_ref, p1_ref, wi_ref, bi_ref, wr1_ref, br1_ref,
               wr2_ref, br2_ref, wd_ref, bd_ref, u_ref, out_ref):
    xv = x_ref[...]
    m = (jnp.dot(xv, wi_ref[...], preferred_element_type=jnp.float32)
         + bi_ref[...] + p0_ref[...] + p1_ref[...])
    h = jnp.dot(m, wr1_ref[...], preferred_element_type=jnp.float32) + br1_ref[...]
    m = m + jnp.dot(h, wr2_ref[...], preferred_element_type=jnp.float32) + br2_ref[...]
    out_ref[...] = (u_ref[...] * xv
                    + jnp.dot(m, wd_ref[...], preferred_element_type=jnp.float32)
                    + bd_ref[...])


def _tc_b(x, p0, p1, wi, bi, wr1, br1, wr2, br2, wd, bd, u):
    grid = (N // _BN,)
    rows = pl.BlockSpec((_BN, F), lambda i: (i, 0))
    full = lambda shape: pl.BlockSpec(shape, lambda i: (0, 0))
    return pl.pallas_call(
        _tc_b_body,
        grid=grid,
        in_specs=[rows, rows, rows,
                  full((F, F)), full((1, F)), full((F, F)), full((1, F)),
                  full((F, F)), full((1, F)), full((F, F)), full((1, F)),
                  full((1, F))],
        out_specs=rows,
        out_shape=jax.ShapeDtypeStruct((N, F), jnp.float32),
    )(x, p0, p1, wi, bi, wr1, br1, wr2, br2, wd, bd, u)


def kernel(x, rbf, idx_i, idx_j, Wk2f, Wi, bi, Wj, bj, Wr1, br1, Wr2, br2,
           Wd, bd, u):
    bi2 = bi.reshape(1, F)
    bj2 = bj.reshape(1, F)
    br12 = br1.reshape(1, F)
    br22 = br2.reshape(1, F)
    bd2 = bd.reshape(1, F)
    u2 = u.reshape(1, F)

    g, dj = _tc_a(rbf.T, x, Wk2f, Wj, bj2)
    partials = _sc_segment_sum(g, idx_i.astype(jnp.int32),
                               idx_j.astype(jnp.int32), dj)
    p0 = partials[:N]
    p1 = partials[N:]
    return _tc_b(x, p0, p1, Wi, bi2, Wr1, br12, Wr2, br22, Wd, bd2, u2)
